# Initial kernel scaffold; baseline (speedup 1.0000x reference)
import jax
import jax.numpy as jnp
from jax.experimental import pallas as pl


def kernel(x, batch, adj_logits, fc1_W, fc1_b, bn1_g, bn1_b, bn1_rm, bn1_rv, fc2_W, fc2_b, bn2_g, bn2_b, bn2_rm, bn2_rv, fc3_W, fc3_b):
    raise NotImplementedError("write your pallas kernel here")


# speedup vs baseline: 10.6274x; 10.6274x over previous
import jax
import jax.numpy as jnp
from jax.experimental import pallas as pl
from jax.experimental.pallas import tpu as pltpu

G = 100
F = 40000
K_TOP = 12000
SR = 8
SC = F // SR
KL = 1000
KS = 8
KT = KS * KL
NKT = F // KT


def _sel_body(adj_ref, t_ref, i_ref):
    a = adj_ref[0]
    bits = jax.lax.bitcast_convert_type(a, jnp.int32) & jnp.int32(0x7FFFFFFF)
    k = jnp.int32(K_TOP)

    def tbody(i, t):
        cand = t | (jnp.int32(1) << (jnp.int32(30) - i))
        cnt = jnp.sum((bits >= cand).astype(jnp.int32))
        return jnp.where(cnt >= k, cand, t)

    t = jax.lax.fori_loop(0, 31, tbody, jnp.int32(0))

    c_gt = jnp.sum((bits > t).astype(jnp.int32))
    m = k - c_gt
    r = jax.lax.broadcasted_iota(jnp.int32, (SR, SC), 0)
    c = jax.lax.broadcasted_iota(jnp.int32, (SR, SC), 1)
    idx = r * jnp.int32(SC) + c
    eq = bits == t

    def ibody(i, cut):
        cand = cut | (jnp.int32(1) << (jnp.int32(16) - i))
        f = jnp.sum((eq & (idx < cand)).astype(jnp.int32))
        return jnp.where(f <= m, cand, cut)

    cut = jax.lax.fori_loop(0, 17, ibody, jnp.int32(0))

    t_ref[...] = jnp.full((1, 1, 128), t, jnp.int32)
    i_ref[...] = jnp.full((1, 1, 128), cut, jnp.int32)


def _mlp_body(adj_ref, x_ref, w1_ref, t_ref, i_ref,
              b1_ref, g1_ref, be1_ref, rm1_ref, rv1_ref,
              w2_ref, b2_ref, g2_ref, be2_ref, rm2_ref, rv2_ref,
              w3_ref, b3_ref, out_ref, acc_ref):
    s = pl.program_id(0)
    bits = jax.lax.bitcast_convert_type(adj_ref[...], jnp.int32) & jnp.int32(0x7FFFFFFF)
    t = t_ref[:, 0:1].reshape(G, 1, 1)
    cut = i_ref[:, 0:1].reshape(G, 1, 1)
    j = jax.lax.broadcasted_iota(jnp.int32, (G, KS, KL), 1)
    c = jax.lax.broadcasted_iota(jnp.int32, (G, KS, KL), 2)
    col = s * jnp.int32(KT) + j * jnp.int32(KL) + c
    mask = (bits > t) | ((bits == t) & (col < cut))
    mask = mask & (bits != jnp.int32(0))
    xm = jnp.where(mask, x_ref[...], jnp.float32(0.0))
    part = jnp.dot(xm[:, 0, :], w1_ref[0], preferred_element_type=jnp.float32)
    for jj in range(1, KS):
        part += jnp.dot(xm[:, jj, :], w1_ref[jj], preferred_element_type=jnp.float32)

    @pl.when(s == 0)
    def _():
        acc_ref[...] = part

    @pl.when(s > 0)
    def _():
        acc_ref[...] += part

    @pl.when(s == NKT - 1)
    def _():
        h = acc_ref[...] + b1_ref[...]
        h = jnp.maximum(h, jnp.float32(0.0))
        h = (h - rm1_ref[...]) * jax.lax.rsqrt(rv1_ref[...] + jnp.float32(1e-5)) \
            * g1_ref[...] + be1_ref[...]
        h = jnp.dot(h, w2_ref[...], preferred_element_type=jnp.float32) + b2_ref[...]
        h = jnp.maximum(h, jnp.float32(0.0))
        h = (h - rm2_ref[...]) * jax.lax.rsqrt(rv2_ref[...] + jnp.float32(1e-5)) \
            * g2_ref[...] + be2_ref[...]
        logits = jnp.dot(h, w3_ref[...], preferred_element_type=jnp.float32) + b3_ref[...]
        mx = jnp.max(logits, axis=-1, keepdims=True)
        lse = jnp.log(jnp.sum(jnp.exp(logits - mx), axis=-1, keepdims=True))
        out_ref[...] = (logits - mx) - lse


def _select_thresholds(adj_flat, interpret=False):
    adj3 = adj_flat.reshape(G, SR, SC)
    t, cut = pl.pallas_call(
        _sel_body,
        grid=(G,),
        in_specs=[pl.BlockSpec((1, SR, SC), lambda g: (g, 0, 0))],
        out_specs=[pl.BlockSpec((1, 1, 128), lambda g: (g, 0, 0)),
                   pl.BlockSpec((1, 1, 128), lambda g: (g, 0, 0))],
        out_shape=[jax.ShapeDtypeStruct((G, 1, 128), jnp.int32),
                   jax.ShapeDtypeStruct((G, 1, 128), jnp.int32)],
        interpret=interpret,
    )(adj3)
    return t.reshape(G, 128), cut.reshape(G, 128)


def _mlp(adj_flat, x_flat, t, cut, fc1_W, fc1_b, bn1_g, bn1_b, bn1_rm, bn1_rv,
         fc2_W, fc2_b, bn2_g, bn2_b, bn2_rm, bn2_rv, fc3_W, fc3_b,
         interpret=False):
    row = lambda a: a.reshape(1, -1)
    const2 = lambda shape: pl.BlockSpec(shape, lambda s: (0, 0))
    return pl.pallas_call(
        _mlp_body,
        grid=(NKT,),
        in_specs=[
            pl.BlockSpec((G, KS, KL), lambda s: (0, s, 0)),
            pl.BlockSpec((G, KS, KL), lambda s: (0, s, 0)),
            pl.BlockSpec((KS, KL, 512), lambda s: (s, 0, 0)),
            const2((G, 128)),
            const2((G, 128)),
            const2((1, 512)), const2((1, 512)), const2((1, 512)),
            const2((1, 512)), const2((1, 512)),
            const2((512, 1024)), const2((1, 1024)), const2((1, 1024)),
            const2((1, 1024)), const2((1, 1024)), const2((1, 1024)),
            const2((1024, 2)), const2((1, 2)),
        ],
        out_specs=pl.BlockSpec((G, 2), lambda s: (0, 0)),
        out_shape=jax.ShapeDtypeStruct((G, 2), jnp.float32),
        scratch_shapes=[pltpu.VMEM((G, 512), jnp.float32)],
        interpret=interpret,
    )(adj_flat.reshape(G, F // KL, KL), x_flat.reshape(G, F // KL, KL),
      fc1_W.reshape(F // KL, KL, 512), t, cut,
      row(fc1_b), row(bn1_g), row(bn1_b), row(bn1_rm), row(bn1_rv),
      fc2_W, row(fc2_b), row(bn2_g), row(bn2_b), row(bn2_rm), row(bn2_rv),
      fc3_W, row(fc3_b))


def kernel(x, batch, adj_logits, fc1_W, fc1_b, bn1_g, bn1_b, bn1_rm, bn1_rv,
           fc2_W, fc2_b, bn2_g, bn2_b, bn2_rm, bn2_rv, fc3_W, fc3_b):
    del batch
    adj_flat = adj_logits.reshape(G, F)
    x_flat = x.reshape(G, F)
    t, cut = _select_thresholds(adj_flat)
    return _mlp(adj_flat, x_flat, t, cut, fc1_W, fc1_b,
                bn1_g, bn1_b, bn1_rm, bn1_rv,
                fc2_W, fc2_b, bn2_g, bn2_b, bn2_rm, bn2_rv, fc3_W, fc3_b)


# speedup vs baseline: 18.2707x; 1.7192x over previous
import functools

import jax
import jax.numpy as jnp
from jax import lax
from jax.experimental import pallas as pl
from jax.experimental.pallas import tpu as pltpu
from jax.experimental.pallas import tpu_sc as plsc

G = 100
F = 40000
K_TOP = 12000
SR = 8
SC = F // SR
KL = 1000
KS = 8
KT = KS * KL
NKT = F // KT


_L = 16


def _sc_select(adj_flat):
    info = plsc.get_sparse_core_info()
    NC, NS = info.num_cores, info.num_subcores
    NW = NC * NS
    NV = F // _L
    mesh = plsc.VectorSubcoreMesh(core_axis_name="c", subcore_axis_name="s")

    @functools.partial(
        pl.kernel, mesh=mesh,
        compiler_params=pltpu.CompilerParams(needs_layout_passes=False),
        out_type=jax.ShapeDtypeStruct((G, _L), jnp.int32),
        scratch_types=[
            pltpu.VMEM((F,), jnp.int32),
            pltpu.VMEM((F,), jnp.int32),
            pltpu.VMEM((F,), jnp.int32),
            pltpu.VMEM((16 * 256,), jnp.int32),
            pltpu.VMEM((256,), jnp.int32),
            pltpu.VMEM((_L,), jnp.int32),
        ],
    )
    def sel(adj_hbm, out_hbm, data_v, cb_v, ci_v, hist_v, comb_v, outv_v):
        wid = lax.axis_index("s") * NC + lax.axis_index("c")
        lanes = lax.broadcasted_iota(jnp.int32, (_L,), 0)
        ones = jnp.ones((_L,), jnp.int32)

        def extract(v, j):
            return jnp.sum(jnp.where(lanes == j, v, jnp.int32(0)))

        def zero_hist():
            def zb(i, c):
                hist_v[pl.ds(i * _L, _L)] = jnp.zeros((_L,), jnp.int32)
                return c
            lax.fori_loop(0, (16 * 256) // _L, zb, 0)

        def combine():
            def cb(d16, c):
                acc = jnp.zeros((_L,), jnp.int32)
                for lane in range(16):
                    acc = acc + hist_v[pl.ds(lane * 256 + d16 * _L, _L)]
                comb_v[pl.ds(d16 * _L, _L)] = acc
                return c
            lax.fori_loop(0, 256 // _L, cb, 0)

        def scan_find(kt):
            def sb(ii, c):
                carry, D, s_next, nb, found = c
                d16 = jnp.int32(15) - ii
                blk = comb_v[pl.ds(d16 * _L, _L)]
                sfx = carry + lax.rev(plsc.cumsum(lax.rev(blk, (0,))), (0,))
                tot = carry + jnp.sum(blk)
                msk = sfx >= kt
                jstar = jnp.sum(msk.astype(jnp.int32)) - 1
                nb_c = extract(blk, jstar)
                sfx_j = extract(sfx, jstar)
                hit = jnp.logical_and(found == 0, tot >= kt)
                D = jnp.where(hit, d16 * _L + jstar, D)
                s_next = jnp.where(hit, sfx_j - nb_c, s_next)
                nb = jnp.where(hit, nb_c, nb)
                found = jnp.where(tot >= kt, jnp.int32(1), found)
                return (tot, D, s_next, nb, found)
            z = jnp.int32(0)
            _, D, s_next, nb, _ = lax.fori_loop(0, 16, sb, (z, z, z, z, z))
            return D, s_next, nb

        def digit1(b):
            return lax.shift_right_logical(b, 23)

        def digit2(b):
            return lax.shift_right_logical(b, 15) & jnp.int32(0xFF)

        def digit3(b):
            return lax.shift_right_logical(b, 7) & jnp.int32(0xFF)

        def one_graph(g):
            pltpu.sync_copy(adj_hbm.at[g], data_v)
            k = jnp.int32(K_TOP)

            zero_hist()

            def r1(i, c):
                b = data_v[pl.ds(i * _L, _L)] & jnp.int32(0x7FFFFFFF)
                plsc.addupdate_scatter(hist_v, [lanes * 256 + digit1(b)], ones)
                return c
            lax.fori_loop(0, NV, r1, 0)
            combine()
            D1, S1, _ = scan_find(k)
            rem2 = k - S1

            zero_hist()

            def r2(i, off):
                b = data_v[pl.ds(i * _L, _L)] & jnp.int32(0x7FFFFFFF)
                sel_m = digit1(b) == D1
                plsc.addupdate_scatter(hist_v, [lanes * 256 + digit2(b)],
                                       ones, mask=sel_m)
                cs = plsc.cumsum(sel_m.astype(jnp.int32))
                pos = off + cs - 1
                plsc.store_scatter(cb_v, [pos], b, mask=sel_m)
                plsc.store_scatter(ci_v, [pos], i * _L + lanes, mask=sel_m)
                return off + jnp.sum(sel_m.astype(jnp.int32))
            n2 = lax.fori_loop(0, NV, r2, jnp.int32(0))
            combine()
            D2, S2, _ = scan_find(rem2)
            rem3 = rem2 - S2
            nv2 = lax.shift_right_logical(n2 + jnp.int32(_L - 1), 4)

            zero_hist()

            def r3(i, c):
                valid = (i * _L + lanes) < n2
                b = cb_v[pl.ds(i * _L, _L)]
                sel_m = valid & (digit2(b) == D2)
                plsc.addupdate_scatter(hist_v, [lanes * 256 + digit3(b)],
                                       ones, mask=sel_m)
                return c
            lax.fori_loop(0, nv2, r3, 0)
            combine()
            D3, S3, _ = scan_find(rem3)
            rem4 = rem3 - S3

            zero_hist()

            def r4(i, c):
                valid = (i * _L + lanes) < n2
                b = cb_v[pl.ds(i * _L, _L)]
                sel_m = valid & (digit2(b) == D2) & (digit3(b) == D3)
                plsc.addupdate_scatter(hist_v, [lanes * 256 + (b & jnp.int32(0x7F))],
                                       ones, mask=sel_m)
                return c
            lax.fori_loop(0, nv2, r4, 0)
            combine()
            D4, S4, _ = scan_find(rem4)
            m = rem4 - S4
            t = (D1 << 23) | (D2 << 15) | (D3 << 7) | D4

            def r5(i, c):
                cnt, cut, found = c
                valid = (i * _L + lanes) < n2
                b = cb_v[pl.ds(i * _L, _L)]
                sel_m = valid & (b == t)
                seli = sel_m.astype(jnp.int32)
                cs = cnt + plsc.cumsum(seli)
                hitl = sel_m & (cs == m)
                has = jnp.sum(hitl.astype(jnp.int32)) > 0
                idxv = ci_v[pl.ds(i * _L, _L)]
                cand = jnp.sum(jnp.where(hitl, idxv, jnp.int32(0))) + 1
                cut = jnp.where(jnp.logical_and(found == 0, has), cand, cut)
                found = jnp.where(has, jnp.int32(1), found)
                return (cnt + jnp.sum(seli), cut, found)
            _, cut, _ = lax.fori_loop(0, nv2, r5,
                                      (jnp.int32(0), jnp.int32(F), jnp.int32(0)))

            outv_v[...] = jnp.where(lanes == 0, t,
                                    jnp.where(lanes == 1, cut, jnp.int32(0)))
            pltpu.sync_copy(outv_v, out_hbm.at[g])

        for gi in range((G + NW - 1) // NW):
            g = wid + NW * gi

            @pl.when(g < G)
            def _():
                one_graph(g)

    return sel(adj_flat)


def _mlp_body(adj_ref, x_ref, w1_ref, tc_ref,
              b1_ref, g1_ref, be1_ref, rm1_ref, rv1_ref,
              w2_ref, b2_ref, g2_ref, be2_ref, rm2_ref, rv2_ref,
              w3_ref, b3_ref, out_ref, acc_ref):
    s = pl.program_id(0)
    bits = jax.lax.bitcast_convert_type(adj_ref[...], jnp.int32) & jnp.int32(0x7FFFFFFF)
    t = tc_ref[:, 0:1].reshape(G, 1, 1)
    cut = tc_ref[:, 1:2].reshape(G, 1, 1)
    j = jax.lax.broadcasted_iota(jnp.int32, (G, KS, KL), 1)
    c = jax.lax.broadcasted_iota(jnp.int32, (G, KS, KL), 2)
    col = s * jnp.int32(KT) + j * jnp.int32(KL) + c
    mask = (bits > t) | ((bits == t) & (col < cut))
    mask = mask & (bits != jnp.int32(0))
    xm = jnp.where(mask, x_ref[...], jnp.float32(0.0))
    part = jnp.dot(xm[:, 0, :], w1_ref[0], preferred_element_type=jnp.float32)
    for jj in range(1, KS):
        part += jnp.dot(xm[:, jj, :], w1_ref[jj], preferred_element_type=jnp.float32)

    @pl.when(s == 0)
    def _():
        acc_ref[...] = part

    @pl.when(s > 0)
    def _():
        acc_ref[...] += part

    @pl.when(s == NKT - 1)
    def _():
        h = acc_ref[...] + b1_ref[...]
        h = jnp.maximum(h, jnp.float32(0.0))
        h = (h - rm1_ref[...]) * jax.lax.rsqrt(rv1_ref[...] + jnp.float32(1e-5)) \
            * g1_ref[...] + be1_ref[...]
        h = jnp.dot(h, w2_ref[...], preferred_element_type=jnp.float32) + b2_ref[...]
        h = jnp.maximum(h, jnp.float32(0.0))
        h = (h - rm2_ref[...]) * jax.lax.rsqrt(rv2_ref[...] + jnp.float32(1e-5)) \
            * g2_ref[...] + be2_ref[...]
        logits = jnp.dot(h, w3_ref[...], preferred_element_type=jnp.float32) + b3_ref[...]
        mx = jnp.max(logits, axis=-1, keepdims=True)
        lse = jnp.log(jnp.sum(jnp.exp(logits - mx), axis=-1, keepdims=True))
        out_ref[...] = (logits - mx) - lse


def _mlp(adj_flat, x_flat, tc, fc1_W, fc1_b, bn1_g, bn1_b, bn1_rm, bn1_rv,
         fc2_W, fc2_b, bn2_g, bn2_b, bn2_rm, bn2_rv, fc3_W, fc3_b,
         interpret=False):
    row = lambda a: a.reshape(1, -1)
    const2 = lambda shape: pl.BlockSpec(shape, lambda s: (0, 0))
    return pl.pallas_call(
        _mlp_body,
        grid=(NKT,),
        in_specs=[
            pl.BlockSpec((G, KS, KL), lambda s: (0, s, 0)),
            pl.BlockSpec((G, KS, KL), lambda s: (0, s, 0)),
            pl.BlockSpec((KS, KL, 512), lambda s: (s, 0, 0)),
            const2((G, _L)),
            const2((1, 512)), const2((1, 512)), const2((1, 512)),
            const2((1, 512)), const2((1, 512)),
            const2((512, 1024)), const2((1, 1024)), const2((1, 1024)),
            const2((1, 1024)), const2((1, 1024)), const2((1, 1024)),
            const2((1024, 2)), const2((1, 2)),
        ],
        out_specs=pl.BlockSpec((G, 2), lambda s: (0, 0)),
        out_shape=jax.ShapeDtypeStruct((G, 2), jnp.float32),
        scratch_shapes=[pltpu.VMEM((G, 512), jnp.float32)],
        interpret=interpret,
    )(adj_flat.reshape(G, F // KL, KL), x_flat.reshape(G, F // KL, KL),
      fc1_W.reshape(F // KL, KL, 512), tc,
      row(fc1_b), row(bn1_g), row(bn1_b), row(bn1_rm), row(bn1_rv),
      fc2_W, row(fc2_b), row(bn2_g), row(bn2_b), row(bn2_rm), row(bn2_rv),
      fc3_W, row(fc3_b))


def kernel(x, batch, adj_logits, fc1_W, fc1_b, bn1_g, bn1_b, bn1_rm, bn1_rv,
           fc2_W, fc2_b, bn2_g, bn2_b, bn2_rm, bn2_rv, fc3_W, fc3_b):
    del batch
    adj_flat = adj_logits.reshape(G, F)
    x_flat = x.reshape(G, F)
    tc = _sc_select(jax.lax.bitcast_convert_type(adj_flat, jnp.int32))
    return _mlp(adj_flat, x_flat, tc, fc1_W, fc1_b,
                bn1_g, bn1_b, bn1_rm, bn1_rv,
                fc2_W, fc2_b, bn2_g, bn2_b, bn2_rm, bn2_rv, fc3_W, fc3_b)
